# baseline (speedup 1.0000x reference)
import functools

import jax
import jax.numpy as jnp
from jax import lax
from jax.experimental import pallas as pl
from jax.experimental.pallas import tpu as pltpu
from jax.experimental.pallas import tpu_sc as plsc

_LANES = 16


@functools.lru_cache(maxsize=None)
def _make_sc_encoder(n_rows, vocab, d_model, seq_len, pos_rows):
    info = plsc.get_sparse_core_info()
    nc, ns = info.num_cores, info.num_subcores
    nw = nc * ns
    chunk = 128
    assert n_rows % (nw * chunk) == 0
    per_w = n_rows // nw
    n_chunks = per_w // chunk
    assert d_model % _LANES == 0
    cvecs = d_model // _LANES

    mesh = plsc.VectorSubcoreMesh(core_axis_name="c", subcore_axis_name="s")

    @functools.partial(
        pl.kernel,
        mesh=mesh,
        out_type=jax.ShapeDtypeStruct((n_rows, d_model), jnp.float32),
        scratch_types=[
            pltpu.VMEM((chunk,), jnp.int32),
            pltpu.VMEM((chunk, d_model), jnp.float32),
            pltpu.VMEM((pos_rows, d_model), jnp.float32),
            pltpu.SemaphoreType.DMA,
        ],
        compiler_params=pltpu.CompilerParams(use_tc_tiling_on_sc=False),
    )
    def enc(tok_hbm, tbl_hbm, pos_hbm, out_hbm, idx_v, rows_v, pos_v, gsem):
        wid = lax.axis_index("s") * nc + lax.axis_index("c")
        base = wid * per_w
        pltpu.sync_copy(pos_hbm, pos_v)

        def chunk_body(g, carry):
            cb = base + g * chunk
            pltpu.sync_copy(tok_hbm.at[pl.ds(cb, chunk)], idx_v)
            pltpu.async_copy(tbl_hbm.at[idx_v], rows_v, gsem).wait()

            def row_body(r, c2):
                p = lax.rem(cb + r, seq_len)
                for c in range(cvecs):
                    sl = pl.ds(c * _LANES, _LANES)
                    rows_v[r, sl] = rows_v[r, sl] + pos_v[p, sl]
                return c2

            lax.fori_loop(0, chunk, row_body, 0)
            pltpu.sync_copy(rows_v, out_hbm.at[pl.ds(cb, chunk)])
            return carry

        lax.fori_loop(0, n_chunks, chunk_body, 0)

    return enc


def kernel(token_ids, token_embed, pos_embed):
    b, s = token_ids.shape
    vocab, d = token_embed.shape
    pos_rows = pos_embed.shape[0]
    tok_flat = token_ids.reshape(-1).astype(jnp.int32)
    enc = _make_sc_encoder(b * s, vocab, d, s, pos_rows)
    out = enc(tok_flat, token_embed, pos_embed)
    return out.reshape(b, s, d)


# speedup vs baseline: 1.2373x; 1.2373x over previous
import functools

import jax
import jax.numpy as jnp
from jax import lax
from jax.experimental import pallas as pl
from jax.experimental.pallas import tpu as pltpu
from jax.experimental.pallas import tpu_sc as plsc

_LANES = 16
_CH = 128
_NBUF = 4


@functools.lru_cache(maxsize=None)
def _make_sc_encoder(n_rows, d_model, seq_len, pos_rows):
    info = plsc.get_sparse_core_info()
    nc, ns = info.num_cores, info.num_subcores
    nw = nc * ns
    assert n_rows % (nw * _CH) == 0
    per_w = n_rows // nw
    n_chunks = per_w // _CH
    assert n_chunks >= 2 * _NBUF and (n_chunks - 2 * _NBUF) % _NBUF == 0
    assert d_model % _LANES == 0
    cvecs = d_model // _LANES

    mesh = plsc.VectorSubcoreMesh(core_axis_name="c", subcore_axis_name="s")

    scratch = [
        pltpu.VMEM((n_chunks, _CH), jnp.int32),
        pltpu.VMEM((pos_rows, d_model), jnp.float32),
    ]
    scratch += [pltpu.VMEM((_CH, d_model), jnp.float32) for _ in range(_NBUF)]
    scratch += [pltpu.VMEM((_CH, d_model), jnp.float32) for _ in range(_NBUF)]
    scratch += [pltpu.SemaphoreType.DMA for _ in range(2 * _NBUF)]

    @functools.partial(
        pl.kernel,
        mesh=mesh,
        out_type=jax.ShapeDtypeStruct((n_rows, d_model), jnp.float32),
        scratch_types=scratch,
        compiler_params=pltpu.CompilerParams(use_tc_tiling_on_sc=False),
    )
    def enc(tok_hbm, tbl_hbm, pos_hbm, out_hbm, idx_v, pos_v, *bufs):
        bins = bufs[:_NBUF]
        bouts = bufs[_NBUF:2 * _NBUF]
        gsems = bufs[2 * _NBUF:3 * _NBUF]
        ssems = bufs[3 * _NBUF:4 * _NBUF]

        wid = lax.axis_index("s") * nc + lax.axis_index("c")
        base = wid * per_w

        pltpu.sync_copy(pos_hbm, pos_v)
        pltpu.sync_copy(tok_hbm.at[pl.ds(wid * n_chunks, n_chunks)], idx_v)

        def fire_gather(g, bb):
            pltpu.async_copy(tbl_hbm.at[idx_v.at[g]], bins[bb], gsems[bb])

        def wait_gather(bb):
            pltpu.make_async_copy(tbl_hbm.at[idx_v.at[0]], bins[bb], gsems[bb]).wait()

        def fire_scatter(g, bb):
            cb = base + g * _CH
            pltpu.async_copy(bouts[bb], out_hbm.at[pl.ds(cb, _CH)], ssems[bb])

        def wait_scatter(bb):
            pltpu.make_async_copy(
                bouts[bb], out_hbm.at[pl.ds(base, _CH)], ssems[bb]).wait()

        def add_pos(g, bb):
            cb = base + g * _CH
            bin_ref, bout_ref = bins[bb], bouts[bb]

            def rows(i, carry):
                r0 = i * 4
                for dr in range(4):
                    r = r0 + dr
                    p = lax.rem(cb + r, seq_len)
                    for c in range(cvecs):
                        sl = pl.ds(c * _LANES, _LANES)
                        bout_ref[r, sl] = bin_ref[r, sl] + pos_v[p, sl]
                return carry

            lax.fori_loop(0, _CH // 4, rows, 0)

        for bb in range(_NBUF):
            fire_gather(bb, bb)
        for bb in range(_NBUF):
            wait_gather(bb)
            add_pos(bb, bb)
            fire_scatter(bb, bb)
            fire_gather(bb + _NBUF, bb)

        def outer(o, carry):
            g0 = _NBUF + o * _NBUF
            for bb in range(_NBUF):
                g = g0 + bb
                wait_gather(bb)
                wait_scatter(bb)
                add_pos(g, bb)
                fire_scatter(g, bb)
                fire_gather(g + _NBUF, bb)
            return carry

        lax.fori_loop(0, (n_chunks - 2 * _NBUF) // _NBUF, outer, 0)

        for bb in range(_NBUF):
            g = n_chunks - _NBUF + bb
            wait_gather(bb)
            wait_scatter(bb)
            add_pos(g, bb)
            fire_scatter(g, bb)
        for bb in range(_NBUF):
            wait_scatter(bb)

    return enc


def kernel(token_ids, token_embed, pos_embed):
    b, s = token_ids.shape
    _, d = token_embed.shape
    pos_rows = pos_embed.shape[0]
    n_rows = b * s
    tok2d = token_ids.reshape(n_rows // _CH, _CH).astype(jnp.int32)
    enc = _make_sc_encoder(n_rows, d, s, pos_rows)
    out = enc(tok2d, token_embed, pos_embed)
    return out.reshape(b, s, d)
